# Initial kernel scaffold; baseline (speedup 1.0000x reference)
import jax
import jax.numpy as jnp
from jax.experimental import pallas as pl


def kernel(x, edge_index, Wl0, Wr0, b0, Wl1, Wr1, b1, Wl2, Wr2, b2, fcW, fcb):
    raise NotImplementedError("write your pallas kernel here")


# speedup vs baseline: 4.7558x; 4.7558x over previous
import functools

import jax
import jax.numpy as jnp
from jax import lax
from jax.experimental import pallas as pl
from jax.experimental.pallas import tpu as pltpu
from jax.experimental.pallas import tpu_sc as plsc

_N = 10000
_E = 320000
_D = 128
_C = 16

_NC = 2
_NS = 16
_NW = _NC * _NS
_CH = 128
_CPW = 79
_EPAD = _NW * _CPW * _CH
_NPAD = 10240
_RPT = _NPAD // _NS


def _agg_body(x_hbm, src_hbm, dst_hbm, out_hbm, src_v, dst_v, rows_v, acc):
    cid = lax.axis_index("c")
    sid = lax.axis_index("s")
    wid = sid * _NC + cid

    zero16 = jnp.zeros((16,), jnp.float32)

    def _zrow(i, carry):
        for c in range(_D // 16):
            rows_v[i, pl.ds(c * 16, 16)] = zero16
        return carry

    lax.fori_loop(0, _CH, _zrow, 0)
    for k in range(_RPT // _CH):
        pltpu.sync_copy(rows_v, acc.at[pl.ds(sid * _RPT + k * _CH, _CH)])
    plsc.subcore_barrier()

    pltpu.sync_copy(src_hbm.at[wid], src_v)
    pltpu.sync_copy(dst_hbm.at[wid], dst_v)

    def _chunk(j, carry):
        pltpu.sync_copy(x_hbm.at[src_v.at[j]], rows_v)
        pltpu.sync_copy(rows_v, acc.at[dst_v.at[j]], add=True)
        return carry

    lax.fori_loop(0, _CPW, _chunk, 0)
    plsc.subcore_barrier()

    pltpu.sync_copy(acc.at[pl.ds(sid * _RPT, _RPT)],
                    out_hbm.at[pl.ds(cid * _NPAD + sid * _RPT, _RPT)])


def _deg_body(dst_hbm, out_hbm, dst_v, ones_v, acc):
    cid = lax.axis_index("c")
    sid = lax.axis_index("s")
    wid = sid * _NC + cid

    zero16 = jnp.zeros((16,), jnp.float32)
    one16 = jnp.ones((16,), jnp.float32)

    def _zrow(i, carry):
        for c in range(_D // 16):
            ones_v[i, pl.ds(c * 16, 16)] = zero16
        return carry

    lax.fori_loop(0, _CH, _zrow, 0)
    for k in range(_RPT // _CH):
        pltpu.sync_copy(ones_v, acc.at[pl.ds(sid * _RPT + k * _CH, _CH)])

    def _orow(i, carry):
        for c in range(_D // 16):
            ones_v[i, pl.ds(c * 16, 16)] = one16
        return carry

    lax.fori_loop(0, _CH, _orow, 0)
    plsc.subcore_barrier()

    pltpu.sync_copy(dst_hbm.at[wid], dst_v)

    def _chunk(j, carry):
        pltpu.sync_copy(ones_v, acc.at[dst_v.at[j]], add=True)
        return carry

    lax.fori_loop(0, _CPW, _chunk, 0)
    plsc.subcore_barrier()

    pltpu.sync_copy(acc.at[pl.ds(sid * _RPT, _RPT)],
                    out_hbm.at[pl.ds(cid * _NPAD + sid * _RPT, _RPT)])


_SC_MESH = plsc.VectorSubcoreMesh(core_axis_name="c", subcore_axis_name="s")

_agg_call = pl.kernel(
    _agg_body,
    out_type=jax.ShapeDtypeStruct((_NC * _NPAD, _D), jnp.float32),
    mesh=_SC_MESH,
    scratch_types=[
        pltpu.VMEM((_CPW, _CH), jnp.int32),
        pltpu.VMEM((_CPW, _CH), jnp.int32),
        pltpu.VMEM((_CH, _D), jnp.float32),
        pltpu.VMEM_SHARED((_NPAD, _D), jnp.float32),
    ],
)

_deg_call = pl.kernel(
    _deg_body,
    out_type=jax.ShapeDtypeStruct((_NC * _NPAD, _D), jnp.float32),
    mesh=_SC_MESH,
    scratch_types=[
        pltpu.VMEM((_CPW, _CH), jnp.int32),
        pltpu.VMEM((_CH, _D), jnp.float32),
        pltpu.VMEM_SHARED((_NPAD, _D), jnp.float32),
    ],
)


_BN = 1000


def _dense_body(relu, d0r, d1r, p0r, p1r, xr, wlr, wrr, br, outr):
    deg = d0r[:, 0:1] + d1r[:, 0:1]
    inv = 1.0 / jnp.maximum(deg, 1.0)
    mean = (p0r[...] + p1r[...]) * inv
    h = (jnp.dot(mean, wlr[...], preferred_element_type=jnp.float32)
         + jnp.dot(xr[...], wrr[...], preferred_element_type=jnp.float32)
         + br[...])
    outr[...] = jnp.maximum(h, 0.0) if relu else h


def _final_body(d0r, d1r, p0r, p1r, xr, wlr, wrr, br, fcwr, fcbr, embr, lsmr):
    deg = d0r[:, 0:1] + d1r[:, 0:1]
    inv = 1.0 / jnp.maximum(deg, 1.0)
    mean = (p0r[...] + p1r[...]) * inv
    emb = (jnp.dot(mean, wlr[...], preferred_element_type=jnp.float32)
           + jnp.dot(xr[...], wrr[...], preferred_element_type=jnp.float32)
           + br[...])
    embr[...] = emb
    logits = jnp.dot(emb, fcwr[...], preferred_element_type=jnp.float32) + fcbr[...]
    m = jnp.max(logits, axis=1, keepdims=True)
    e = jnp.exp(logits - m)
    lsmr[...] = (logits - m) - jnp.log(jnp.sum(e, axis=1, keepdims=True))


def _row_spec(w):
    return pl.BlockSpec((_BN, w), lambda i: (i, 0))


def _full_spec(h, w):
    return pl.BlockSpec((h, w), lambda i: (0, 0))


def _dense_layer(relu, d0, d1, p0, p1, x, Wl, Wr, b):
    return pl.pallas_call(
        functools.partial(_dense_body, relu),
        grid=(_N // _BN,),
        in_specs=[_row_spec(_D), _row_spec(_D), _row_spec(_D), _row_spec(_D),
                  _row_spec(_D), _full_spec(_D, _D), _full_spec(_D, _D),
                  _full_spec(1, _D)],
        out_specs=_row_spec(_D),
        out_shape=jax.ShapeDtypeStruct((_N, _D), jnp.float32),
    )(d0, d1, p0, p1, x, Wl, Wr, b.reshape(1, _D))


def _final_layer(d0, d1, p0, p1, x, Wl, Wr, b, fcW, fcb):
    return pl.pallas_call(
        _final_body,
        grid=(_N // _BN,),
        in_specs=[_row_spec(_D), _row_spec(_D), _row_spec(_D), _row_spec(_D),
                  _row_spec(_D), _full_spec(_D, _D), _full_spec(_D, _D),
                  _full_spec(1, _D), _full_spec(_D, _C), _full_spec(1, _C)],
        out_specs=[_row_spec(_D), _row_spec(_C)],
        out_shape=[jax.ShapeDtypeStruct((_N, _D), jnp.float32),
                   jax.ShapeDtypeStruct((_N, _C), jnp.float32)],
    )(d0, d1, p0, p1, x, Wl, Wr, b.reshape(1, _D), fcW, fcb.reshape(1, _C))


def kernel(x, edge_index, Wl0, Wr0, b0, Wl1, Wr1, b1, Wl2, Wr2, b2, fcW, fcb):
    src = edge_index[0]
    dst = edge_index[1]
    pad = _EPAD - _E
    srcp = jnp.concatenate([src, jnp.zeros((pad,), jnp.int32)]).reshape(_NW, _CPW, _CH)
    dstp = jnp.concatenate([dst, jnp.full((pad,), _N, jnp.int32)]).reshape(_NW, _CPW, _CH)

    dp = _deg_call(dstp)
    d0 = dp[:_N]
    d1 = dp[_NPAD:_NPAD + _N]

    h = x
    for li, (Wl, Wr, b) in enumerate(((Wl0, Wr0, b0), (Wl1, Wr1, b1), (Wl2, Wr2, b2))):
        p = _agg_call(h, srcp, dstp)
        p0 = p[:_N]
        p1 = p[_NPAD:_NPAD + _N]
        if li < 2:
            h = _dense_layer(True, d0, d1, p0, p1, h, Wl, Wr, b)
        else:
            emb, lsm = _final_layer(d0, d1, p0, p1, h, Wl, Wr, b, fcW, fcb)
    return (emb, lsm)
